# Initial kernel scaffold; baseline (speedup 1.0000x reference)
import jax
import jax.numpy as jnp
from jax.experimental import pallas as pl


def kernel(pos, edge_index, W_init, freqs, W_rbf, W_msg, W_upd, W_out):
    raise NotImplementedError("write your pallas kernel here")


# speedup vs baseline: 2.7572x; 2.7572x over previous
import functools

import jax
import jax.numpy as jnp
from jax import lax
from jax.experimental import pallas as pl
from jax.experimental.pallas import tpu as pltpu
from jax.experimental.pallas import tpu_sc as plsc

DIM = 128
N_NODES = 10000
N_EDGES = 320000
N_RBF = 16
CUTOFF_G = 10.0
ENV_EXP = 5
OUT_DIM = 15

NC = 2
NS = 16
NW = NC * NS
LANES = 16


_EPT_D = N_EDGES // NW


def _dist2_body(px_h, py_h, pz_h, src_h, dst_h, d2_h,
                px_v, py_v, pz_v, src_v, dst_v, d2_v):
    c = lax.axis_index("c")
    s = lax.axis_index("s")
    wid = s * NC + c
    base = wid * _EPT_D
    pltpu.sync_copy(px_h, px_v)
    pltpu.sync_copy(py_h, py_v)
    pltpu.sync_copy(pz_h, pz_v)
    pltpu.sync_copy(src_h.at[pl.ds(base, _EPT_D)], src_v)
    pltpu.sync_copy(dst_h.at[pl.ds(base, _EPT_D)], dst_v)

    def step(i, _):
        sl = pl.ds(i * LANES, LANES)
        si = src_v[sl]
        di = dst_v[sl]
        dx = plsc.load_gather(px_v, [di]) - plsc.load_gather(px_v, [si])
        dy = plsc.load_gather(py_v, [di]) - plsc.load_gather(py_v, [si])
        dz = plsc.load_gather(pz_v, [di]) - plsc.load_gather(pz_v, [si])
        d2_v[sl] = dx * dx + dy * dy + dz * dz
        return 0

    lax.fori_loop(0, _EPT_D // LANES, step, 0)
    pltpu.sync_copy(d2_v, d2_h.at[pl.ds(base, _EPT_D)])


def _dist2_call(px, py, pz, src, dst):
    mesh = plsc.VectorSubcoreMesh(core_axis_name="c", subcore_axis_name="s",
                                  num_cores=NC, num_subcores=NS)
    return pl.kernel(
        _dist2_body,
        out_type=jax.ShapeDtypeStruct((N_EDGES,), jnp.float32),
        mesh=mesh,
        compiler_params=pltpu.CompilerParams(needs_layout_passes=False),
        scratch_types=[
            pltpu.VMEM((N_NODES,), jnp.float32),
            pltpu.VMEM((N_NODES,), jnp.float32),
            pltpu.VMEM((N_NODES,), jnp.float32),
            pltpu.VMEM((_EPT_D,), jnp.int32),
            pltpu.VMEM((_EPT_D,), jnp.int32),
            pltpu.VMEM((_EPT_D,), jnp.float32),
        ],
    )(px, py, pz, src, dst)


_EPC = N_EDGES // NC
_EPT = _EPC // NS
_K = 80
_NCHUNK = _EPT // _K
_ZROWS = 80
_NZCH = N_NODES // _ZROWS


def _gms_body(x_h, ew_h, src_h, dst_h, out_h,
              rows_v, ew_v, prod_v, src_i, dst_i, zbuf, acc, sem):
    c = lax.axis_index("c")
    s = lax.axis_index("s")

    def zrow(i, _):
        for j in range(DIM // LANES):
            zbuf[i, pl.ds(j * LANES, LANES)] = jnp.zeros((LANES,), jnp.float32)
        return 0

    lax.fori_loop(0, _ZROWS, zrow, 0)

    def zcopy(i, _):
        ch = s + i * NS

        @pl.when(ch < _NZCH)
        def _():
            pltpu.sync_copy(zbuf, acc.at[pl.ds(ch * _ZROWS, _ZROWS)])
        return 0

    lax.fori_loop(0, (_NZCH + NS - 1) // NS, zcopy, 0)
    plsc.subcore_barrier()

    ebase = c * _EPC + s * _EPT

    def chunk(i, _):
        eb = ebase + i * _K
        pltpu.sync_copy(src_h.at[pl.ds(eb, _K)], src_i)
        pltpu.sync_copy(dst_h.at[pl.ds(eb, _K)], dst_i)
        pltpu.sync_copy(ew_h.at[pl.ds(eb, _K)], ew_v)
        pltpu.async_copy(x_h.at[src_i], rows_v, sem).wait()

        def mul_row(r, _):
            for j in range(DIM // LANES):
                fsl = pl.ds(j * LANES, LANES)
                prod_v[r, fsl] = rows_v[r, fsl] * ew_v[r, fsl]
            return 0

        lax.fori_loop(0, _K, mul_row, 0)
        pltpu.sync_copy(prod_v, acc.at[dst_i], add=True)
        return 0

    lax.fori_loop(0, _NCHUNK, chunk, 0)
    plsc.subcore_barrier()

    def rback(i, _):
        ch = s + i * NS

        @pl.when(ch < _NZCH)
        def _():
            r = ch * _ZROWS
            pltpu.sync_copy(acc.at[pl.ds(r, _ZROWS)], zbuf)
            pltpu.sync_copy(zbuf, out_h.at[c, pl.ds(r, _ZROWS)])
        return 0

    lax.fori_loop(0, (_NZCH + NS - 1) // NS, rback, 0)


def _gather_mul_scatter(x, ew, src, dst):
    mesh = plsc.VectorSubcoreMesh(core_axis_name="c", subcore_axis_name="s",
                                  num_cores=NC, num_subcores=NS)
    return pl.kernel(
        _gms_body,
        out_type=jax.ShapeDtypeStruct((NC, N_NODES, DIM), jnp.float32),
        mesh=mesh,
        compiler_params=pltpu.CompilerParams(needs_layout_passes=False),
        scratch_types=[
            pltpu.VMEM((_K, DIM), jnp.float32),
            pltpu.VMEM((_K, DIM), jnp.float32),
            pltpu.VMEM((_K, DIM), jnp.float32),
            pltpu.VMEM((_K,), jnp.int32),
            pltpu.VMEM((_K,), jnp.int32),
            pltpu.VMEM((_ZROWS, DIM), jnp.float32),
            pltpu.VMEM_SHARED((N_NODES, DIM), jnp.float32),
            pltpu.SemaphoreType.DMA,
        ],
    )(x, ew, src, dst)


_BN = 1000


def _init_tc_body(pos_ref, w_ref, x_ref):
    p = pos_ref[...]
    w = w_ref[...]
    acc = p[:, 0:1] * w[0:1, :]
    acc += p[:, 1:2] * w[1:2, :]
    acc += p[:, 2:3] * w[2:3, :]
    x_ref[...] = jnp.maximum(acc, 0.0)


def _init_tc(pos, W_init):
    return pl.pallas_call(
        _init_tc_body,
        grid=(N_NODES // _BN,),
        in_specs=[
            pl.BlockSpec((_BN, 3), lambda i: (i, 0)),
            pl.BlockSpec((3, DIM), lambda i: (0, 0)),
        ],
        out_specs=pl.BlockSpec((_BN, DIM), lambda i: (i, 0)),
        out_shape=jax.ShapeDtypeStruct((N_NODES, DIM), jnp.float32),
    )(pos, W_init)


_BE = 2560


def _ew_tc_body(d2_ref, freqs_ref, wrbf_ref, ew_ref):
    d2 = d2_ref[...]
    dist = jnp.sqrt(d2 + 1e-12)
    d = dist * (1.0 / CUTOFF_G)
    p = ENV_EXP + 1
    a = -(p + 1) * (p + 2) / 2.0
    b = p * (p + 2)
    cc = -p * (p + 1) / 2.0
    d_safe = jnp.maximum(d, 1e-6)
    env = 1.0 / d_safe + a * d_safe ** (p - 1) + b * d_safe ** p \
        + cc * d_safe ** (p + 1)
    env = jnp.where(d < 1.0, env, 0.0)
    rbf = env * jnp.sin(d * freqs_ref[...])
    ew_ref[...] = jnp.maximum(
        jnp.dot(rbf, wrbf_ref[...], preferred_element_type=jnp.float32), 0.0)


def _ew_tc(d2, freqs_row, W_rbf):
    return pl.pallas_call(
        _ew_tc_body,
        grid=(N_EDGES // _BE,),
        in_specs=[
            pl.BlockSpec((_BE, 1), lambda i: (i, 0)),
            pl.BlockSpec((1, N_RBF), lambda i: (0, 0)),
            pl.BlockSpec((N_RBF, DIM), lambda i: (0, 0)),
        ],
        out_specs=pl.BlockSpec((_BE, DIM), lambda i: (i, 0)),
        out_shape=jax.ShapeDtypeStruct((N_EDGES, DIM), jnp.float32),
    )(d2, freqs_row, W_rbf)


def _upd_tc_body(x_ref, g_ref, wm_ref, wu_ref, xo_ref):
    gsum = g_ref[0] + g_ref[1]
    aggr = jnp.dot(gsum, wm_ref[...], preferred_element_type=jnp.float32)
    h = jnp.dot(aggr, wu_ref[...], preferred_element_type=jnp.float32)
    xo_ref[...] = jnp.maximum(x_ref[...] + h, 0.0)


def _upd_tc(x, g, Wm, Wu):
    return pl.pallas_call(
        _upd_tc_body,
        grid=(N_NODES // _BN,),
        in_specs=[
            pl.BlockSpec((_BN, DIM), lambda i: (i, 0)),
            pl.BlockSpec((NC, _BN, DIM), lambda i: (0, i, 0)),
            pl.BlockSpec((DIM, DIM), lambda i: (0, 0)),
            pl.BlockSpec((DIM, DIM), lambda i: (0, 0)),
        ],
        out_specs=pl.BlockSpec((_BN, DIM), lambda i: (i, 0)),
        out_shape=jax.ShapeDtypeStruct((N_NODES, DIM), jnp.float32),
    )(x, g, Wm, Wu)


def _final_tc_body(x_ref, g_ref, wm_ref, wu_ref, wo_ref, out_ref):
    gsum = g_ref[0] + g_ref[1]
    aggr = jnp.dot(gsum, wm_ref[...], preferred_element_type=jnp.float32)
    h = jnp.dot(aggr, wu_ref[...], preferred_element_type=jnp.float32)
    x2 = jnp.maximum(x_ref[...] + h, 0.0)
    out_ref[...] = jnp.dot(x2, wo_ref[...], preferred_element_type=jnp.float32)


def _final_tc(x, g, Wm, Wu, Wo_pad):
    return pl.pallas_call(
        _final_tc_body,
        grid=(N_NODES // _BN,),
        in_specs=[
            pl.BlockSpec((_BN, DIM), lambda i: (i, 0)),
            pl.BlockSpec((NC, _BN, DIM), lambda i: (0, i, 0)),
            pl.BlockSpec((DIM, DIM), lambda i: (0, 0)),
            pl.BlockSpec((DIM, DIM), lambda i: (0, 0)),
            pl.BlockSpec((DIM, DIM), lambda i: (0, 0)),
        ],
        out_specs=pl.BlockSpec((_BN, DIM), lambda i: (i, 0)),
        out_shape=jax.ShapeDtypeStruct((N_NODES, DIM), jnp.float32),
    )(x, g, Wm, Wu, Wo_pad)


@jax.jit
def _run(pos, edge_index, W_init, freqs, W_rbf, W_msg, W_upd, W_out):
    src = edge_index[0]
    dst = edge_index[1]
    px = pos[:, 0]
    py = pos[:, 1]
    pz = pos[:, 2]

    d2 = _dist2_call(px, py, pz, src, dst)
    ew = _ew_tc(d2.reshape(N_EDGES, 1), freqs.reshape(1, N_RBF), W_rbf)
    x = _init_tc(pos, W_init)

    g = _gather_mul_scatter(x, ew, src, dst)
    x = _upd_tc(x, g, W_msg[0], W_upd[0])

    g = _gather_mul_scatter(x, ew, src, dst)
    Wo_pad = jnp.pad(W_out, ((0, 0), (0, DIM - OUT_DIM)))
    out = _final_tc(x, g, W_msg[1], W_upd[1], Wo_pad)
    return out[:, :OUT_DIM]


def kernel(pos, edge_index, W_init, freqs, W_rbf, W_msg, W_upd, W_out):
    return _run(pos, edge_index, W_init, freqs, W_rbf, W_msg, W_upd, W_out)
